# Initial kernel scaffold; baseline (speedup 1.0000x reference)
import jax
import jax.numpy as jnp
from jax.experimental import pallas as pl

H = 32
D = 64
CS = 256
TOPK = 4
EPS = 1e-6


def _rope(x, pos, base=10000.0):
    d = x.shape[-1]
    inv = 1.0 / (base ** (jnp.arange(0, d, 2, dtype=jnp.float32) / d))
    fr = pos[:, None].astype(jnp.float32) * inv[None, :]
    cos = jnp.cos(fr)[None, :, None, :]
    sin = jnp.sin(fr)[None, :, None, :]
    x1, x2 = x[..., : d // 2], x[..., d // 2 :]
    return jnp.concatenate([x1 * cos - x2 * sin, x2 * cos + x1 * sin], axis=-1)


def kernel(hidden_states, Wq, Wk, Wv, Wo, Wg1, Wg2, o_norm_w):
    HI = jax.lax.Precision.HIGHEST
    B, S, HID = hidden_states.shape
    q = jnp.dot(hidden_states[0], Wq, precision=HI).reshape(B, S, H, D)
    k = jnp.dot(hidden_states[0], Wk, precision=HI).reshape(B, S, H, D)
    v = jnp.dot(hidden_states[0], Wv, precision=HI).reshape(B, S, H, D)
    pos = jnp.arange(S)
    q = _rope(q, pos)
    k = _rope(k, pos)
    C = S // CS
    kc = k.reshape(B, C, CS, H, D).mean(axis=2)
    gate = jnp.einsum('bshd,bchd->bhsc', q, kc, precision=HI)
    cid = pos // CS
    c_idx = jnp.arange(C)
    not_past = pos[:, None] < (c_idx[None, :] + 1) * CS
    self_chunk = cid[:, None] == c_idx[None, :]
    gate = jnp.where(not_past[None, None], -jnp.inf, gate)
    gate = jnp.where(self_chunk[None, None], jnp.inf, gate)
    kk = min(TOPK, C)
    topv, _ = jax.lax.top_k(gate, kk)
    thresh = topv[..., -1:]
    need = gate >= thresh
    attend = jnp.take(need, cid, axis=-1)
    causal = pos[None, :] <= pos[:, None]
    mask = attend & causal[None, None]
    scale = 1.0 / jnp.sqrt(jnp.float32(D))
    qk = jnp.einsum('bshd,bthd->bhst', q, k, precision=HI) * scale
    qk = jnp.where(mask, qk, -jnp.inf)
    p = jax.nn.softmax(qk, axis=-1)
    o = jnp.einsum('bhst,bthd->bshd', p, v, precision=HI)
    g = jnp.dot(jnp.dot(hidden_states[0], Wg1, precision=HI), Wg2, precision=HI).reshape(B, S, H, D)
    rms = jax.lax.rsqrt(jnp.mean(jnp.square(o), axis=-1, keepdims=True) + EPS)
    o = o * rms * o_norm_w * jax.nn.sigmoid(g)
    o = o.reshape(B, S, HID)
    return jnp.dot(o[0], Wo, precision=HI)[None]


# speedup vs baseline: 1.1298x; 1.1298x over previous
import functools

import jax
import jax.numpy as jnp
from jax.experimental import pallas as pl

H = 32
D = 64
CS = 256
TOPK = 4
EPS = 1e-6
NEG = -1e30

BF = jnp.bfloat16
F32 = jnp.float32


def _dot(a, b, dims):
    return jax.lax.dot_general(a, b, dimension_numbers=(dims, ((), ())),
                               preferred_element_type=F32)


def _proj_rope_kern(hs_ref, w_ref, wp_ref, cos_ref, sin_ref, out_ref, f32_ref):
    x = _dot(hs_ref[...], w_ref[...], ((1,), (0,)))
    xp = _dot(hs_ref[...], wp_ref[...], ((1,), (0,)))
    r = x * cos_ref[...] + xp * sin_ref[...]
    out_ref[...] = r.astype(BF)
    if f32_ref is not None:
        f32_ref[...] = r


def _proj_rope(hsb, w, wp, cosT, sinT2, with_f32):
    S, HID = hsb.shape
    HD = w.shape[1]
    BN = 512
    grid = (S // CS, HD // BN)
    out_shape = [jax.ShapeDtypeStruct((S, HD), BF)]
    out_specs = [pl.BlockSpec((CS, BN), lambda r, n: (r, n))]
    if with_f32:
        out_shape.append(jax.ShapeDtypeStruct((S, HD), F32))
        out_specs.append(pl.BlockSpec((CS, BN), lambda r, n: (r, n)))
        kern = _proj_rope_kern
    else:
        kern = lambda *a: _proj_rope_kern(*a, None)
    res = pl.pallas_call(
        kern,
        grid=grid,
        in_specs=[
            pl.BlockSpec((CS, HID), lambda r, n: (r, 0)),
            pl.BlockSpec((HID, BN), lambda r, n: (0, n)),
            pl.BlockSpec((HID, BN), lambda r, n: (0, n)),
            pl.BlockSpec((CS, BN), lambda r, n: (r, n)),
            pl.BlockSpec((CS, BN), lambda r, n: (r, n)),
        ],
        out_specs=out_specs,
        out_shape=out_shape,
    )(hsb, w, wp, cosT, sinT2)
    return res if with_f32 else (res[0], None)


def _proj_kern(hs_ref, w_ref, out_ref):
    out_ref[...] = _dot(hs_ref[...], w_ref[...], ((1,), (0,))).astype(BF)


def _proj(hsb, w):
    S, HID = hsb.shape
    HD = w.shape[1]
    BN = 512
    return pl.pallas_call(
        _proj_kern,
        grid=(S // CS, HD // BN),
        in_specs=[
            pl.BlockSpec((CS, HID), lambda r, n: (r, 0)),
            pl.BlockSpec((HID, BN), lambda r, n: (0, n)),
        ],
        out_specs=pl.BlockSpec((CS, BN), lambda r, n: (r, n)),
        out_shape=jax.ShapeDtypeStruct((S, HD), BF),
    )(hsb, w)


def _gatepath_kern(hs_ref, w1_ref, w2_ref, onw_ref, out_ref):
    t = _dot(hs_ref[...], w1_ref[...], ((1,), (0,))).astype(BF)
    g = _dot(t, w2_ref[...], ((1,), (0,)))
    out_ref[...] = jax.nn.sigmoid(g) * onw_ref[0:1, :]


def _gatepath(hsb, w1, w2, onw_t):
    S, HID = hsb.shape
    return pl.pallas_call(
        _gatepath_kern,
        grid=(S // CS,),
        in_specs=[
            pl.BlockSpec((CS, HID), lambda r: (r, 0)),
            pl.BlockSpec((HID, D), lambda r: (0, 0)),
            pl.BlockSpec((D, HID), lambda r: (0, 0)),
            pl.BlockSpec((8, HID), lambda r: (0, 0)),
        ],
        out_specs=pl.BlockSpec((CS, HID), lambda r: (r, 0)),
        out_shape=jax.ShapeDtypeStruct((S, HID), F32),
    )(hsb, w1, w2, onw_t)


def _moba_kern(qb_ref, kf_ref, bias_ref, *, S, C):
    kf = kf_ref[0]
    kc = jnp.concatenate(
        [jnp.mean(kf[c * CS:(c + 1) * CS, :], axis=0, keepdims=True)
         for c in range(C)], axis=0)
    kcb = kc.astype(BF)
    g = _dot(kcb, qb_ref[0], ((1,), (1,)))
    c = jax.lax.broadcasted_iota(jnp.int32, (C, S), 0)
    pos = jax.lax.broadcasted_iota(jnp.int32, (C, S), 1)
    cid = pos // CS
    not_past = pos < (c + 1) * CS
    g = jnp.where(not_past, -jnp.inf, g)
    g = jnp.where(cid == c, jnp.inf, g)
    ge = g[None, :, :] >= g[:, None, :]
    cnt = jnp.sum(ge.astype(jnp.int32), axis=1)
    cand = jnp.where(cnt >= TOPK, g, -jnp.inf)
    thresh = jnp.max(cand, axis=0, keepdims=True)
    need = g >= thresh
    bias_ref[0] = jnp.where(need, 0.0, NEG).astype(F32)


def _moba_bias(qb3, kf3):
    _, S, _ = qb3.shape
    C = S // CS
    return pl.pallas_call(
        functools.partial(_moba_kern, S=S, C=C),
        grid=(H,),
        in_specs=[
            pl.BlockSpec((1, S, D), lambda h: (h, 0, 0)),
            pl.BlockSpec((1, S, D), lambda h: (h, 0, 0)),
        ],
        out_specs=pl.BlockSpec((1, C, S), lambda h: (h, 0, 0)),
        out_shape=jax.ShapeDtypeStruct((H, C, S), F32),
    )(qb3, kf3)


def _attn_kern(qb_ref, kb_ref, vb_ref, bias_ref, sg_ref, out_ref, *, scale, C):
    i = pl.program_id(1)
    qb = qb_ref[0]
    biasb = bias_ref[0]

    ci = jax.lax.broadcasted_iota(jnp.int32, (CS, CS), 0)
    qi = jax.lax.broadcasted_iota(jnp.int32, (CS, CS), 1)
    csel = jax.lax.broadcasted_iota(jnp.int32, (C, CS), 0)

    def body(j, carry):
        m, l, acc = carry
        kj = kb_ref[0, pl.ds(j * CS, CS), :]
        vj = vb_ref[0, pl.ds(j * CS, CS), :]
        s = _dot(kj, qb, ((1,), (1,))) * scale
        brow = jnp.sum(jnp.where(csel == j, biasb, 0.0), axis=0, keepdims=True)
        s = s + brow
        s = jnp.where(jnp.logical_and(j == i, ci > qi), NEG, s)
        m_new = jnp.maximum(m, jnp.max(s, axis=0, keepdims=True))
        r = jnp.exp(m - m_new)
        p = jnp.exp(s - m_new)
        l = l * r + jnp.sum(p, axis=0, keepdims=True)
        acc = acc * r + _dot(vj, p.astype(BF), ((0,), (0,)))
        return m_new, l, acc

    m0 = jnp.full((1, CS), NEG, dtype=F32)
    l0 = jnp.zeros((1, CS), dtype=F32)
    a0 = jnp.zeros((D, CS), dtype=F32)
    m, l, acc = jax.lax.fori_loop(0, i + 1, body, (m0, l0, a0))

    oT = acc / l
    rms = jax.lax.rsqrt(jnp.mean(oT * oT, axis=0, keepdims=True) + EPS)
    out_ref[...] = (oT * rms * sg_ref[...]).astype(BF)


def _attention(qb3, kb3, vb3, bias3, sgT):
    _, S, _ = qb3.shape
    C = S // CS
    scale = 1.0 / (D ** 0.5)
    return pl.pallas_call(
        functools.partial(_attn_kern, scale=scale, C=C),
        grid=(H, C),
        in_specs=[
            pl.BlockSpec((1, CS, D), lambda h, i: (h, i, 0)),
            pl.BlockSpec((1, S, D), lambda h, i: (h, 0, 0)),
            pl.BlockSpec((1, S, D), lambda h, i: (h, 0, 0)),
            pl.BlockSpec((1, C, CS), lambda h, i: (h, 0, i)),
            pl.BlockSpec((D, CS), lambda h, i: (h, i)),
        ],
        out_specs=pl.BlockSpec((D, CS), lambda h, i: (h, i)),
        out_shape=jax.ShapeDtypeStruct((H * D, S), BF),
    )(qb3, kb3, vb3, bias3, sgT)


def _outproj_kern(oT_ref, w_ref, out_ref):
    out_ref[...] = _dot(oT_ref[...], w_ref[...], ((0,), (0,)))


def _outproj(oT, w):
    HD, S = oT.shape
    HID = w.shape[1]
    BN = 512
    return pl.pallas_call(
        _outproj_kern,
        grid=(S // BN, HID // BN),
        in_specs=[
            pl.BlockSpec((HD, BN), lambda r, n: (0, r)),
            pl.BlockSpec((HD, BN), lambda r, n: (0, n)),
        ],
        out_specs=pl.BlockSpec((BN, BN), lambda r, n: (r, n)),
        out_shape=jax.ShapeDtypeStruct((S, HID), F32),
    )(oT, w)


def _swap_halves(w):
    HID = w.shape[0]
    w3 = w.reshape(HID, H, D)
    return jnp.concatenate([w3[:, :, D // 2:], w3[:, :, : D // 2]],
                           axis=-1).reshape(HID, H * D)


def _headmajor(x):
    S, HD = x.shape
    return x.reshape(S, H, D).transpose(1, 0, 2)


def kernel(hidden_states, Wq, Wk, Wv, Wo, Wg1, Wg2, o_norm_w):
    B, S, HID = hidden_states.shape
    hs = hidden_states[0]
    hsb = hs.astype(BF)

    pos = jnp.arange(S)
    inv = 1.0 / (10000.0 ** (jnp.arange(0, D, 2, dtype=F32) / D))
    fr = pos[:, None].astype(F32) * inv[None, :]
    cos = jnp.cos(fr)
    sin = jnp.sin(fr)
    cosT = jnp.tile(jnp.concatenate([cos, cos], axis=1), (1, H))
    sinT2 = jnp.tile(jnp.concatenate([-sin, sin], axis=1), (1, H))

    qb, _ = _proj_rope(hsb, Wq.astype(BF), _swap_halves(Wq).astype(BF),
                       cosT, sinT2, with_f32=False)
    kb, kf = _proj_rope(hsb, Wk.astype(BF), _swap_halves(Wk).astype(BF),
                        cosT, sinT2, with_f32=True)
    vb = _proj(hsb, Wv.astype(BF))
    sg = _gatepath(hsb, Wg1.astype(BF), Wg2.astype(BF),
                   jnp.broadcast_to(jnp.tile(o_norm_w, H)[None, :], (8, H * D)))
    sgT = sg.T

    qb3 = _headmajor(qb)
    kb3 = _headmajor(kb)
    vb3 = _headmajor(vb)
    kf3 = _headmajor(kf)

    bias3 = _moba_bias(qb3, kf3)
    oT = _attention(qb3, kb3, vb3, bias3, sgT)
    out = _outproj(oT, Wo.astype(BF))
    return out[None]
